# Initial kernel scaffold; baseline (speedup 1.0000x reference)
import jax
import jax.numpy as jnp
from jax.experimental import pallas as pl


def kernel(x, W1, b1, W2, b2, W3, b3, W4, b4, codebook):
    raise NotImplementedError("write your pallas kernel here")


# speedup vs baseline: 4.3927x; 4.3927x over previous
import functools

import jax
import jax.numpy as jnp
from jax.experimental import pallas as pl
from jax.experimental.pallas import tpu as pltpu

B = 16384
EMB = 10
NC = 16
SP = 20
BM = 512


def _vq_body(x_ref, w1_ref, b1_ref, w2_ref, b2_ref, m_ref, c2_ref, p_ref,
             w3_ref, b3_ref, w4_ref, b4_ref, recon_ref, ze_ref, zq_ref):
    h1 = jnp.maximum(
        jnp.dot(x_ref[...], w1_ref[...], preferred_element_type=jnp.float32)
        + b1_ref[...], 0.0)
    h2 = jnp.dot(h1, w2_ref[...], preferred_element_type=jnp.float32) + b2_ref[...]
    ze_ref[...] = h2

    d = jnp.dot(h2, m_ref[...], preferred_element_type=jnp.float32) + c2_ref[...]
    lane = jax.lax.broadcasted_iota(jnp.int32, d.shape, 1)
    off = jnp.bitwise_and(lane, NC - 1)

    gm = d
    for sh in (8, 4, 2, 1):
        fwd = jnp.roll(gm, -sh, axis=1)
        bwd = jnp.roll(gm, sh, axis=1)
        gm = jnp.minimum(gm, jnp.where(jnp.bitwise_and(off, sh) == 0, fwd, bwd))

    cand = jnp.where(d == gm, off, NC)
    for sh in (8, 4, 2, 1):
        fwd = jnp.roll(cand, -sh, axis=1)
        bwd = jnp.roll(cand, sh, axis=1)
        cand = jnp.minimum(cand, jnp.where(jnp.bitwise_and(off, sh) == 0, fwd, bwd))

    onehot = (off == cand).astype(jnp.float32)
    quant = jnp.dot(onehot, p_ref[...], preferred_element_type=jnp.float32)
    zq_ref[...] = quant

    h3 = jnp.maximum(
        jnp.dot(quant, w3_ref[...], preferred_element_type=jnp.float32)
        + b3_ref[...], 0.0)
    recon_ref[...] = jax.nn.sigmoid(
        jnp.dot(h3, w4_ref[...], preferred_element_type=jnp.float32) + b4_ref[...])


@functools.partial(jax.jit, static_argnames=())
def _run(x, W1T, b1, W2T, b2, M, c2, P, W3T, b3, W4T, b4):
    grid = (B // BM,)
    row_blk = lambda shp: pl.BlockSpec(shp, lambda i: (i, 0))
    full_blk = lambda shp: pl.BlockSpec(shp, lambda i: (0, 0))
    out_shapes = (
        jax.ShapeDtypeStruct((B, 784), jnp.float32),
        jax.ShapeDtypeStruct((B, 200), jnp.float32),
        jax.ShapeDtypeStruct((B, 200), jnp.float32),
    )
    return pl.pallas_call(
        _vq_body,
        grid=grid,
        in_specs=[
            row_blk((BM, 784)),
            full_blk((784, 400)), full_blk((1, 400)),
            full_blk((400, 200)), full_blk((1, 200)),
            full_blk((200, SP * NC)), full_blk((1, SP * NC)),
            full_blk((SP * NC, 200)),
            full_blk((200, 400)), full_blk((1, 400)),
            full_blk((400, 784)), full_blk((1, 784)),
        ],
        out_specs=(row_blk((BM, 784)), row_blk((BM, 200)), row_blk((BM, 200))),
        out_shape=out_shapes,
        compiler_params=pltpu.CompilerParams(
            dimension_semantics=("arbitrary",)),
    )(x, W1T, b1, W2T, b2, M, c2, P, W3T, b3, W4T, b4)


def kernel(x, W1, b1, W2, b2, W3, b3, W4, b4, codebook):
    e = jnp.arange(200) // SP
    s = jnp.arange(200) % SP
    M = jnp.zeros((200, SP * NC), jnp.float32)
    cols = s[:, None] * NC + jnp.arange(NC)[None, :]
    M = M.at[jnp.arange(200)[:, None], cols].set(-2.0 * codebook.T[e])
    c2 = jnp.tile(jnp.sum(codebook * codebook, axis=1), SP)[None, :]
    P = jnp.zeros((SP * NC, 200), jnp.float32)
    rows = jnp.arange(SP)[:, None, None] * NC + jnp.arange(NC)[None, :, None]
    pcols = jnp.arange(EMB)[None, None, :] * SP + jnp.arange(SP)[:, None, None]
    P = P.at[jnp.broadcast_to(rows, (SP, NC, EMB)),
             jnp.broadcast_to(pcols, (SP, NC, EMB))].set(
        jnp.broadcast_to(codebook[None], (SP, NC, EMB)))

    recon, ze_flat, zq_flat = _run(
        x, W1.T, b1[None, :], W2.T, b2[None, :], M, c2, P,
        W3.T, b3[None, :], W4.T, b4[None, :])
    return (recon, ze_flat.reshape(B, EMB, SP), zq_flat)


# speedup vs baseline: 5.5487x; 1.2631x over previous
import functools

import jax
import jax.numpy as jnp
from jax.experimental import pallas as pl
from jax.experimental.pallas import tpu as pltpu

B = 16384
EMB = 10
NC = 16
SP = 20
SPPAD = 24
BM = 512


def _vq_body(x_ref, w1_ref, b1_ref, w2_ref, b2_ref, m_ref, c2_ref, p_ref,
             w3_ref, b3_ref, w4_ref, b4_ref, recon_ref, ze_ref, zq_ref):
    h1 = jnp.maximum(
        jnp.dot(x_ref[...], w1_ref[...], preferred_element_type=jnp.float32)
        + b1_ref[...], 0.0)
    h2 = jnp.dot(h1, w2_ref[...], preferred_element_type=jnp.float32) + b2_ref[...]
    ze_ref[...] = h2

    d = jnp.dot(h2, m_ref[...], preferred_element_type=jnp.float32) + c2_ref[...]
    bm = d.shape[0]
    off = jnp.bitwise_and(
        jax.lax.broadcasted_iota(jnp.int32, (bm, 128), 1), NC - 1)

    parts = []
    for t in range(3):
        dt = jax.lax.slice(d, (0, t * 128), (bm, (t + 1) * 128))
        bits = jax.lax.bitcast_convert_type(dt, jnp.int32)
        key = jax.lax.bitwise_xor(
            bits, jax.lax.bitwise_and(
                jax.lax.shift_right_arithmetic(bits, 31),
                jnp.int32(0x7FFFFFFF)))
        keyk = jax.lax.bitwise_or(
            jax.lax.bitwise_and(key, jnp.int32(~(NC - 1))), off)
        for sh in (8, 4, 2, 1):
            fwd = jnp.roll(keyk, -sh, axis=1)
            bwd = jnp.roll(keyk, sh, axis=1)
            keyk = jnp.minimum(
                keyk, jnp.where(jnp.bitwise_and(off, sh) == 0, fwd, bwd))
        win = jnp.bitwise_and(keyk, NC - 1)
        parts.append((off == win).astype(jnp.float32))
    onehot = jnp.concatenate(parts, axis=1)
    quant = jnp.dot(onehot, p_ref[...], preferred_element_type=jnp.float32)
    zq_ref[...] = quant

    h3 = jnp.maximum(
        jnp.dot(quant, w3_ref[...], preferred_element_type=jnp.float32)
        + b3_ref[...], 0.0)
    recon_ref[...] = jax.nn.sigmoid(
        jnp.dot(h3, w4_ref[...], preferred_element_type=jnp.float32) + b4_ref[...])


@functools.partial(jax.jit, static_argnames=())
def _run(x, W1T, b1, W2T, b2, M, c2, P, W3T, b3, W4T, b4):
    grid = (B // BM,)
    row_blk = lambda shp: pl.BlockSpec(shp, lambda i: (i, 0))
    full_blk = lambda shp: pl.BlockSpec(shp, lambda i: (0, 0))
    out_shapes = (
        jax.ShapeDtypeStruct((B, 784), jnp.float32),
        jax.ShapeDtypeStruct((B, 200), jnp.float32),
        jax.ShapeDtypeStruct((B, 200), jnp.float32),
    )
    return pl.pallas_call(
        _vq_body,
        grid=grid,
        in_specs=[
            row_blk((BM, 784)),
            full_blk((784, 400)), full_blk((1, 400)),
            full_blk((400, 200)), full_blk((1, 200)),
            full_blk((200, SPPAD * NC)), full_blk((1, SPPAD * NC)),
            full_blk((SPPAD * NC, 200)),
            full_blk((200, 400)), full_blk((1, 400)),
            full_blk((400, 784)), full_blk((1, 784)),
        ],
        out_specs=(row_blk((BM, 784)), row_blk((BM, 200)), row_blk((BM, 200))),
        out_shape=out_shapes,
        compiler_params=pltpu.CompilerParams(
            dimension_semantics=("arbitrary",)),
    )(x, W1T, b1, W2T, b2, M, c2, P, W3T, b3, W4T, b4)


def kernel(x, W1, b1, W2, b2, W3, b3, W4, b4, codebook):
    e = jnp.arange(200) // SP
    s = jnp.arange(200) % SP
    gcol = (s // 8) * 128 + (s % 8) * NC
    M = jnp.zeros((200, SPPAD * NC), jnp.float32)
    cols = gcol[:, None] + jnp.arange(NC)[None, :]
    M = M.at[jnp.arange(200)[:, None], cols].set(-2.0 * codebook.T[e])
    c2 = jnp.tile(jnp.sum(codebook * codebook, axis=1), SPPAD)[None, :]
    P = jnp.zeros((SPPAD * NC, 200), jnp.float32)
    grow = (jnp.arange(SP) // 8) * 128 + (jnp.arange(SP) % 8) * NC
    rows = grow[:, None, None] + jnp.arange(NC)[None, :, None]
    pcols = jnp.arange(EMB)[None, None, :] * SP + jnp.arange(SP)[:, None, None]
    P = P.at[jnp.broadcast_to(rows, (SP, NC, EMB)),
             jnp.broadcast_to(pcols, (SP, NC, EMB))].set(
        jnp.broadcast_to(codebook[None], (SP, NC, EMB)))

    recon, ze_flat, zq_flat = _run(
        x, W1.T, b1[None, :], W2.T, b2[None, :], M, c2, P,
        W3.T, b3[None, :], W4.T, b4[None, :])
    return (recon, ze_flat.reshape(B, EMB, SP), zq_flat)
